# baseline (speedup 1.0000x reference)
import jax
import jax.numpy as jnp
from jax import lax
from jax.experimental import pallas as pl
from jax.experimental.pallas import tpu as pltpu
from jax.experimental.pallas import tpu_sc as plsc

_N, _E, _D = 10000, 320000, 128
_NC, _NS = 2, 16
_NW = _NC * _NS
_CH = 32
_KF = 488
_KS = 144
_FAST_C = 0
_EPWF = _KF * _CH
_EPWS = _KS * _CH
_EPAD = _NS * (_EPWF + _EPWS)
_NACC = _N + 16
_DCH = 80
_DNCH = (_E // _NW) // _DCH
_RPT = _N // _NS
_B = 2000
_G = _N // _B


def _deg_body(dst_hbm, out_hbm, dst_v, deg_v):
    c = lax.axis_index("c")
    s = lax.axis_index("s")
    wid = c * _NS + s
    pltpu.sync_copy(dst_hbm.at[wid], dst_v)
    zero16 = jnp.zeros((16,), jnp.float32)

    def zb(i, carry):
        deg_v[pl.ds(i * 16, 16)] = zero16
        return carry

    lax.fori_loop(0, _N // 16, zb, 0)

    ones16 = jnp.ones((16,), jnp.float32)
    lanes = lax.iota(jnp.int32, 16)

    def chunk(i, carry):
        def inner(k, c2):
            idx = dst_v[i, pl.ds(k * 16, 16)]
            for lane in range(16):
                plsc.addupdate_scatter(deg_v, [idx], ones16,
                                       mask=lanes == lane)
            return c2

        return lax.fori_loop(0, _DCH // 16, inner, carry)

    lax.fori_loop(0, _DNCH, chunk, 0)
    pltpu.sync_copy(deg_v, out_hbm.at[pl.ds(wid * _N, _N)])


def _deg_counts(dst3):
    kf = pl.kernel(
        _deg_body,
        out_type=jax.ShapeDtypeStruct((_NW * _N,), jnp.float32),
        mesh=plsc.VectorSubcoreMesh(core_axis_name="c", subcore_axis_name="s"),
        scratch_types=[
            pltpu.VMEM((_DNCH, _DCH), jnp.int32),
            pltpu.VMEM((_N,), jnp.float32),
        ],
        compiler_params=pltpu.CompilerParams(needs_layout_passes=False),
    )
    return kf(dst3)


def _edge_body(g_hbm, src_hbm, dst_hbm, out_hbm,
               src_v, dst_v, stage0, rows0, rows1, rows2, rows3, acc_sh,
               semg0, semg1, semg2, semg3):
    c = lax.axis_index("c")
    s = lax.axis_index("s")
    is_fast = c == _FAST_C
    base = jnp.where(is_fast, s * _EPWF, _NS * _EPWF + s * _EPWS)
    kc = jnp.where(is_fast, _KF, _KS)

    @pl.when(is_fast)
    def _():
        pltpu.sync_copy(src_hbm.at[pl.ds(base, _EPWF)], src_v)
        pltpu.sync_copy(dst_hbm.at[pl.ds(base, _EPWF)], dst_v)

    @pl.when(jnp.logical_not(is_fast))
    def _():
        pltpu.sync_copy(src_hbm.at[pl.ds(base, _EPWS)],
                        src_v.at[pl.ds(0, _EPWS)])
        pltpu.sync_copy(dst_hbm.at[pl.ds(base, _EPWS)],
                        dst_v.at[pl.ds(0, _EPWS)])

    zero16 = jnp.zeros((16,), jnp.float32)

    def zrow(i, carry):
        def zcol(j, c2):
            rows0[i, pl.ds(j * 16, 16)] = zero16
            return c2

        return lax.fori_loop(0, _D // 16, zcol, carry)

    lax.fori_loop(0, _CH, zrow, 0)
    for k in range(_RPT // _CH):
        pltpu.sync_copy(rows0, acc_sh.at[pl.ds(s * _RPT + k * _CH, _CH)])
    _TAIL = _RPT - (_RPT // _CH) * _CH
    pltpu.sync_copy(rows0.at[pl.ds(0, _TAIL)],
                    acc_sh.at[pl.ds(s * _RPT + (_RPT // _CH) * _CH, _TAIL)])
    plsc.subcore_barrier()

    rows = (rows0, rows1, rows2, rows3)
    gsems = (semg0, semg1, semg2, semg3)

    def issue_gather(j, slot):
        pltpu.async_copy(g_hbm.at[src_v.at[pl.ds(j * _CH, _CH)]],
                         rows[slot], gsems[slot])

    issue_gather(0, 0)
    issue_gather(1, 1)
    issue_gather(2, 2)

    def step(j, b):
        pltpu.make_async_copy(g_hbm.at[src_v.at[pl.ds(j * _CH, _CH)]],
                              rows[b], gsems[b]).wait()
        for k in range(_CH // 16):
            stage0[pl.ds(k * 16, 16)] = dst_v[pl.ds(j * _CH + k * 16, 16)]
        pltpu.sync_copy(rows[b], acc_sh.at[stage0], add=True)

        @pl.when(j + 3 < kc)
        def _():
            issue_gather(j + 3, (b + 3) % 4)

    def quad(r, carry):
        j = 4 * r
        for b in range(4):
            step(j + b, b)
        return carry

    lax.fori_loop(0, kc // 4, quad, 0)

    plsc.subcore_barrier()
    base = s * 624
    pltpu.sync_copy(acc_sh.at[pl.ds(base, 624)],
                    out_hbm.at[c, pl.ds(base, 624)])

    @pl.when(s == _NS - 1)
    def _():
        pltpu.sync_copy(acc_sh.at[pl.ds(_NS * 624, _N - _NS * 624)],
                        out_hbm.at[c, pl.ds(_NS * 624, _N - _NS * 624)])


def _edge_pass(g, srcp, dstp):
    kf = pl.kernel(
        _edge_body,
        out_type=jax.ShapeDtypeStruct((_NC, _N, _D), jnp.float32),
        mesh=plsc.VectorSubcoreMesh(core_axis_name="c", subcore_axis_name="s"),
        scratch_types=(
            [pltpu.VMEM((_EPWF,), jnp.int32),
             pltpu.VMEM((_EPWF,), jnp.int32),
             pltpu.VMEM((_CH,), jnp.int32)]
            + [pltpu.VMEM((_CH, _D), jnp.float32) for _ in range(4)]
            + [pltpu.VMEM_SHARED((_NACC, _D), jnp.float32)]
            + [pltpu.SemaphoreType.DMA for _ in range(4)]
        ),
    )
    return kf(g, srcp, dstp)


def _dinv_body(degp_ref, dinv_ref):
    deg = jnp.sum(degp_ref[...], axis=0) + 1.0
    dinv_ref[...] = lax.rsqrt(deg).reshape(_N, 1)


def _tc1_body(x_ref, w_ref, dinv_ref, g_ref):
    h = jnp.dot(x_ref[...], w_ref[...], preferred_element_type=jnp.float32)
    g_ref[...] = h * dinv_ref[...]


def _tc2_body(acc_ref, g1_ref, dinv_ref, b1_ref, w2_ref, g2_ref):
    dinv = dinv_ref[...]
    acc = acc_ref[0] + acc_ref[1]
    z = (acc + g1_ref[...]) * dinv + b1_ref[...]
    o = jnp.maximum(z, 0.0)
    h2 = jnp.dot(o, w2_ref[...], preferred_element_type=jnp.float32)
    g2_ref[...] = h2 * dinv


def _tc3_body(acc_ref, g2_ref, dinv_ref, b2_ref, wp_ref, bp_ref,
              h_ref, vals_ref, idx_ref):
    i = pl.program_id(0)
    dinv = dinv_ref[...]
    acc = acc_ref[0] + acc_ref[1]
    h = (acc + g2_ref[...]) * dinv + b2_ref[...]
    h_ref[...] = h
    pge = jnp.dot(h, wp_ref[...], preferred_element_type=jnp.float32) + bp_ref[...]
    m = jnp.max(pge, axis=0)[None, :]
    rows = lax.broadcasted_iota(jnp.int32, pge.shape, 0)
    am = jnp.min(jnp.where(pge == m, rows, _N), axis=0)[None, :] + i * _B

    @pl.when(i == 0)
    def _():
        vals_ref[...] = m
        idx_ref[...] = am

    @pl.when(i > 0)
    def _():
        cur = vals_ref[...]
        upd = m > cur
        vals_ref[...] = jnp.where(upd, m, cur)
        idx_ref[...] = jnp.where(upd, am, idx_ref[...])


def _row_spec():
    return pl.BlockSpec((_B, _D), lambda i: (i, 0))


def _full_spec(shape):
    nd = len(shape)
    return pl.BlockSpec(shape, lambda i: (0,) * nd)


def _dinv_spec():
    return pl.BlockSpec((_B, 1), lambda i: (i, 0))


def _acc_spec():
    return pl.BlockSpec((_NC, _B, _D), lambda i: (0, i, 0))


def kernel(x, edge_index, W1, b1, W2, b2, Wp, bp):
    src, dst = edge_index[0], edge_index[1]
    dst3 = dst.reshape(_NW, _DNCH, _DCH)
    pad = _EPAD - _E
    srcp = jnp.concatenate([src, jnp.zeros((pad,), jnp.int32)])
    dstp = jnp.concatenate([dst, jnp.full((pad,), _N, jnp.int32)])
    b1r = b1.reshape(1, _D)
    b2r = b2.reshape(1, _D)
    bpr = bp.reshape(1, _D)

    degp = _deg_counts(dst3).reshape(_NW, _N)

    dinv = pl.pallas_call(
        _dinv_body,
        grid=(1,),
        in_specs=[_full_spec((_NW, _N))],
        out_specs=_full_spec((_N, 1)),
        out_shape=jax.ShapeDtypeStruct((_N, 1), jnp.float32),
    )(degp)

    g1 = pl.pallas_call(
        _tc1_body,
        grid=(_G,),
        in_specs=[_row_spec(), _full_spec((_D, _D)), _dinv_spec()],
        out_specs=_row_spec(),
        out_shape=jax.ShapeDtypeStruct((_N, _D), jnp.float32),
    )(x, W1, dinv)

    acc1 = _edge_pass(g1, srcp, dstp)

    g2 = pl.pallas_call(
        _tc2_body,
        grid=(_G,),
        in_specs=[_acc_spec(), _row_spec(), _dinv_spec(),
                  _full_spec((1, _D)), _full_spec((_D, _D))],
        out_specs=_row_spec(),
        out_shape=jax.ShapeDtypeStruct((_N, _D), jnp.float32),
    )(acc1, g1, dinv, b1r, W2)

    acc2 = _edge_pass(g2, srcp, dstp)

    h, vals, idx = pl.pallas_call(
        _tc3_body,
        grid=(_G,),
        in_specs=[_acc_spec(), _row_spec(), _dinv_spec(),
                  _full_spec((1, _D)), _full_spec((_D, _D)),
                  _full_spec((1, _D))],
        out_specs=[_row_spec(), _full_spec((1, _D)), _full_spec((1, _D))],
        out_shape=[jax.ShapeDtypeStruct((_N, _D), jnp.float32),
                   jax.ShapeDtypeStruct((1, _D), jnp.float32),
                   jax.ShapeDtypeStruct((1, _D), jnp.int32)],
    )(acc2, g2, dinv, b2r, Wp, bpr)

    return h, vals.reshape(_D), idx.reshape(_D)


# speedup vs baseline: 1.0089x; 1.0089x over previous
import jax
import jax.numpy as jnp
from jax import lax
from jax.experimental import pallas as pl
from jax.experimental.pallas import tpu as pltpu
from jax.experimental.pallas import tpu_sc as plsc

_N, _E, _D = 10000, 320000, 128
_NC, _NS = 2, 16
_NW = _NC * _NS
_CH = 64
_KF = 244
_KS = 72
_FAST_C = 0
_EPWF = _KF * _CH
_EPWS = _KS * _CH
_EPAD = _NS * (_EPWF + _EPWS)
_IMASK = (1 << 14) - 1
_NACC = _N + 16
_DCH = 80
_DNCH = (_E // _NW) // _DCH
_RPT = _N // _NS
_B = 2000
_G = _N // _B


def _deg_body(dst_hbm, out_hbm, dst_v, deg_v):
    c = lax.axis_index("c")
    s = lax.axis_index("s")
    wid = c * _NS + s
    pltpu.sync_copy(dst_hbm.at[wid], dst_v)
    zero16 = jnp.zeros((16,), jnp.float32)

    def zb(i, carry):
        deg_v[pl.ds(i * 16, 16)] = zero16
        return carry

    lax.fori_loop(0, _N // 16, zb, 0)

    ones16 = jnp.ones((16,), jnp.float32)
    lanes = lax.iota(jnp.int32, 16)

    def chunk(i, carry):
        def inner(k, c2):
            idx = dst_v[i, pl.ds(k * 16, 16)]
            for lane in range(16):
                plsc.addupdate_scatter(deg_v, [idx], ones16,
                                       mask=lanes == lane)
            return c2

        return lax.fori_loop(0, _DCH // 16, inner, carry)

    lax.fori_loop(0, _DNCH, chunk, 0)
    pltpu.sync_copy(deg_v, out_hbm.at[pl.ds(wid * _N, _N)])


def _deg_counts(dst3):
    kf = pl.kernel(
        _deg_body,
        out_type=jax.ShapeDtypeStruct((_NW * _N,), jnp.float32),
        mesh=plsc.VectorSubcoreMesh(core_axis_name="c", subcore_axis_name="s"),
        scratch_types=[
            pltpu.VMEM((_DNCH, _DCH), jnp.int32),
            pltpu.VMEM((_N,), jnp.float32),
        ],
        compiler_params=pltpu.CompilerParams(needs_layout_passes=False),
    )
    return kf(dst3)


def _edge_body(g_hbm, e_hbm, out_hbm,
               ei_v, sstage0, sstage1, sstage2, sstage3, dstage,
               rows0, rows1, rows2, rows3, acc_sh,
               semg0, semg1, semg2, semg3):
    c = lax.axis_index("c")
    s = lax.axis_index("s")
    is_fast = c == _FAST_C
    base = jnp.where(is_fast, s * _EPWF, _NS * _EPWF + s * _EPWS)
    kc = jnp.where(is_fast, _KF, _KS)

    @pl.when(is_fast)
    def _():
        pltpu.sync_copy(e_hbm.at[pl.ds(base, _EPWF)], ei_v)

    @pl.when(jnp.logical_not(is_fast))
    def _():
        pltpu.sync_copy(e_hbm.at[pl.ds(base, _EPWS)],
                        ei_v.at[pl.ds(0, _EPWS)])

    zero16 = jnp.zeros((16,), jnp.float32)

    def zrow(i, carry):
        def zcol(j, c2):
            rows0[i, pl.ds(j * 16, 16)] = zero16
            return c2

        return lax.fori_loop(0, _D // 16, zcol, carry)

    lax.fori_loop(0, _CH, zrow, 0)
    for k in range(_RPT // _CH):
        pltpu.sync_copy(rows0, acc_sh.at[pl.ds(s * _RPT + k * _CH, _CH)])
    _TAIL = _RPT - (_RPT // _CH) * _CH
    pltpu.sync_copy(rows0.at[pl.ds(0, _TAIL)],
                    acc_sh.at[pl.ds(s * _RPT + (_RPT // _CH) * _CH, _TAIL)])
    plsc.subcore_barrier()

    rows = (rows0, rows1, rows2, rows3)
    sstages = (sstage0, sstage1, sstage2, sstage3)
    gsems = (semg0, semg1, semg2, semg3)

    def issue_gather(j, slot):
        for k in range(_CH // 16):
            p = ei_v[pl.ds(j * _CH + k * 16, 16)]
            sstages[slot][pl.ds(k * 16, 16)] = p & _IMASK
        pltpu.async_copy(g_hbm.at[sstages[slot]], rows[slot], gsems[slot])

    issue_gather(0, 0)
    issue_gather(1, 1)
    issue_gather(2, 2)

    def step(j, b):
        pltpu.make_async_copy(g_hbm.at[sstages[b]], rows[b],
                              gsems[b]).wait()
        for k in range(_CH // 16):
            p = ei_v[pl.ds(j * _CH + k * 16, 16)]
            dstage[pl.ds(k * 16, 16)] = lax.shift_right_logical(p, 14)
        pltpu.sync_copy(rows[b], acc_sh.at[dstage], add=True)

        @pl.when(j + 3 < kc)
        def _():
            issue_gather(j + 3, (b + 3) % 4)

    def quad(r, carry):
        j = 4 * r
        for b in range(4):
            step(j + b, b)
        return carry

    lax.fori_loop(0, kc // 4, quad, 0)

    plsc.subcore_barrier()
    base = s * 624
    pltpu.sync_copy(acc_sh.at[pl.ds(base, 624)],
                    out_hbm.at[c, pl.ds(base, 624)])

    @pl.when(s == _NS - 1)
    def _():
        pltpu.sync_copy(acc_sh.at[pl.ds(_NS * 624, _N - _NS * 624)],
                        out_hbm.at[c, pl.ds(_NS * 624, _N - _NS * 624)])


def _edge_pass(g, ep):
    kf = pl.kernel(
        _edge_body,
        out_type=jax.ShapeDtypeStruct((_NC, _N, _D), jnp.float32),
        mesh=plsc.VectorSubcoreMesh(core_axis_name="c", subcore_axis_name="s"),
        scratch_types=(
            [pltpu.VMEM((_EPWF,), jnp.int32)]
            + [pltpu.VMEM((_CH,), jnp.int32) for _ in range(5)]
            + [pltpu.VMEM((_CH, _D), jnp.float32) for _ in range(4)]
            + [pltpu.VMEM_SHARED((_NACC, _D), jnp.float32)]
            + [pltpu.SemaphoreType.DMA for _ in range(4)]
        ),
    )
    return kf(g, ep)


def _dinv_body(degp_ref, dinv_ref):
    deg = jnp.sum(degp_ref[...], axis=0) + 1.0
    dinv_ref[...] = lax.rsqrt(deg).reshape(_N, 1)


def _tc1_body(x_ref, w_ref, dinv_ref, g_ref):
    h = jnp.dot(x_ref[...], w_ref[...], preferred_element_type=jnp.float32)
    g_ref[...] = h * dinv_ref[...]


def _tc2_body(acc_ref, g1_ref, dinv_ref, b1_ref, w2_ref, g2_ref):
    dinv = dinv_ref[...]
    acc = acc_ref[0] + acc_ref[1]
    z = (acc + g1_ref[...]) * dinv + b1_ref[...]
    o = jnp.maximum(z, 0.0)
    h2 = jnp.dot(o, w2_ref[...], preferred_element_type=jnp.float32)
    g2_ref[...] = h2 * dinv


def _tc3_body(acc_ref, g2_ref, dinv_ref, b2_ref, wp_ref, bp_ref,
              h_ref, vals_ref, idx_ref):
    i = pl.program_id(0)
    dinv = dinv_ref[...]
    acc = acc_ref[0] + acc_ref[1]
    h = (acc + g2_ref[...]) * dinv + b2_ref[...]
    h_ref[...] = h
    pge = jnp.dot(h, wp_ref[...], preferred_element_type=jnp.float32) + bp_ref[...]
    m = jnp.max(pge, axis=0)[None, :]
    rows = lax.broadcasted_iota(jnp.int32, pge.shape, 0)
    am = jnp.min(jnp.where(pge == m, rows, _N), axis=0)[None, :] + i * _B

    @pl.when(i == 0)
    def _():
        vals_ref[...] = m
        idx_ref[...] = am

    @pl.when(i > 0)
    def _():
        cur = vals_ref[...]
        upd = m > cur
        vals_ref[...] = jnp.where(upd, m, cur)
        idx_ref[...] = jnp.where(upd, am, idx_ref[...])


def _row_spec():
    return pl.BlockSpec((_B, _D), lambda i: (i, 0))


def _full_spec(shape):
    nd = len(shape)
    return pl.BlockSpec(shape, lambda i: (0,) * nd)


def _dinv_spec():
    return pl.BlockSpec((_B, 1), lambda i: (i, 0))


def _acc_spec():
    return pl.BlockSpec((_NC, _B, _D), lambda i: (0, i, 0))


def kernel(x, edge_index, W1, b1, W2, b2, Wp, bp):
    src, dst = edge_index[0], edge_index[1]
    dst3 = dst.reshape(_NW, _DNCH, _DCH)
    pad = _EPAD - _E
    packed = jnp.bitwise_or(src, jnp.left_shift(dst, 14))
    ep = jnp.concatenate([packed, jnp.full((pad,), _N << 14, jnp.int32)])
    b1r = b1.reshape(1, _D)
    b2r = b2.reshape(1, _D)
    bpr = bp.reshape(1, _D)

    degp = _deg_counts(dst3).reshape(_NW, _N)

    dinv = pl.pallas_call(
        _dinv_body,
        grid=(1,),
        in_specs=[_full_spec((_NW, _N))],
        out_specs=_full_spec((_N, 1)),
        out_shape=jax.ShapeDtypeStruct((_N, 1), jnp.float32),
    )(degp)

    g1 = pl.pallas_call(
        _tc1_body,
        grid=(_G,),
        in_specs=[_row_spec(), _full_spec((_D, _D)), _dinv_spec()],
        out_specs=_row_spec(),
        out_shape=jax.ShapeDtypeStruct((_N, _D), jnp.float32),
    )(x, W1, dinv)

    acc1 = _edge_pass(g1, ep)

    g2 = pl.pallas_call(
        _tc2_body,
        grid=(_G,),
        in_specs=[_acc_spec(), _row_spec(), _dinv_spec(),
                  _full_spec((1, _D)), _full_spec((_D, _D))],
        out_specs=_row_spec(),
        out_shape=jax.ShapeDtypeStruct((_N, _D), jnp.float32),
    )(acc1, g1, dinv, b1r, W2)

    acc2 = _edge_pass(g2, ep)

    h, vals, idx = pl.pallas_call(
        _tc3_body,
        grid=(_G,),
        in_specs=[_acc_spec(), _row_spec(), _dinv_spec(),
                  _full_spec((1, _D)), _full_spec((_D, _D)),
                  _full_spec((1, _D))],
        out_specs=[_row_spec(), _full_spec((1, _D)), _full_spec((1, _D))],
        out_shape=[jax.ShapeDtypeStruct((_N, _D), jnp.float32),
                   jax.ShapeDtypeStruct((1, _D), jnp.float32),
                   jax.ShapeDtypeStruct((1, _D), jnp.int32)],
    )(acc2, g2, dinv, b2r, Wp, bpr)

    return h, vals.reshape(_D), idx.reshape(_D)


# speedup vs baseline: 1.0498x; 1.0405x over previous
import jax
import jax.numpy as jnp
from jax import lax
from jax.experimental import pallas as pl
from jax.experimental.pallas import tpu as pltpu
from jax.experimental.pallas import tpu_sc as plsc

_N, _E, _D = 10000, 320000, 128
_NC, _NS = 2, 16
_NW = _NC * _NS
_CH = 64
_KF = 260
_KS = 56
_FAST_C = 0
_EPWF = _KF * _CH
_EPWS = _KS * _CH
_EPAD = _NS * (_EPWF + _EPWS)
_IMASK = (1 << 14) - 1
_NACC = _N + 16
_DCH = 80
_DNCH = (_E // _NW) // _DCH
_RPT = _N // _NS
_B = 2000
_G = _N // _B


def _deg_body(dst_hbm, out_hbm, dst_v, deg_v):
    c = lax.axis_index("c")
    s = lax.axis_index("s")
    wid = c * _NS + s
    pltpu.sync_copy(dst_hbm.at[wid], dst_v)
    zero16 = jnp.zeros((16,), jnp.float32)

    def zb(i, carry):
        deg_v[pl.ds(i * 16, 16)] = zero16
        return carry

    lax.fori_loop(0, _N // 16, zb, 0)

    ones16 = jnp.ones((16,), jnp.float32)
    lanes = lax.iota(jnp.int32, 16)

    def chunk(i, carry):
        def inner(k, c2):
            idx = dst_v[i, pl.ds(k * 16, 16)]
            for lane in range(16):
                plsc.addupdate_scatter(deg_v, [idx], ones16,
                                       mask=lanes == lane)
            return c2

        return lax.fori_loop(0, _DCH // 16, inner, carry)

    lax.fori_loop(0, _DNCH, chunk, 0)
    pltpu.sync_copy(deg_v, out_hbm.at[pl.ds(wid * _N, _N)])


def _deg_counts(dst3):
    kf = pl.kernel(
        _deg_body,
        out_type=jax.ShapeDtypeStruct((_NW * _N,), jnp.float32),
        mesh=plsc.VectorSubcoreMesh(core_axis_name="c", subcore_axis_name="s"),
        scratch_types=[
            pltpu.VMEM((_DNCH, _DCH), jnp.int32),
            pltpu.VMEM((_N,), jnp.float32),
        ],
        compiler_params=pltpu.CompilerParams(needs_layout_passes=False),
    )
    return kf(dst3)


def _edge_body(g_hbm, e_hbm, out_hbm,
               ei_v, sstage0, sstage1, sstage2, sstage3, dstage,
               rows0, rows1, rows2, rows3, acc_sh,
               semg0, semg1, semg2, semg3):
    c = lax.axis_index("c")
    s = lax.axis_index("s")
    is_fast = c == _FAST_C
    base = jnp.where(is_fast, s * _EPWF, _NS * _EPWF + s * _EPWS)
    kc = jnp.where(is_fast, _KF, _KS)

    @pl.when(is_fast)
    def _():
        pltpu.sync_copy(e_hbm.at[pl.ds(base, _EPWF)], ei_v)

    @pl.when(jnp.logical_not(is_fast))
    def _():
        pltpu.sync_copy(e_hbm.at[pl.ds(base, _EPWS)],
                        ei_v.at[pl.ds(0, _EPWS)])

    zero16 = jnp.zeros((16,), jnp.float32)

    def zrow(i, carry):
        def zcol(j, c2):
            rows0[i, pl.ds(j * 16, 16)] = zero16
            return c2

        return lax.fori_loop(0, _D // 16, zcol, carry)

    lax.fori_loop(0, _CH, zrow, 0)
    for k in range(_RPT // _CH):
        pltpu.sync_copy(rows0, acc_sh.at[pl.ds(s * _RPT + k * _CH, _CH)])
    _TAIL = _RPT - (_RPT // _CH) * _CH
    pltpu.sync_copy(rows0.at[pl.ds(0, _TAIL)],
                    acc_sh.at[pl.ds(s * _RPT + (_RPT // _CH) * _CH, _TAIL)])
    plsc.subcore_barrier()

    rows = (rows0, rows1, rows2, rows3)
    sstages = (sstage0, sstage1, sstage2, sstage3)
    gsems = (semg0, semg1, semg2, semg3)

    def issue_gather(j, slot):
        for k in range(_CH // 16):
            p = ei_v[pl.ds(j * _CH + k * 16, 16)]
            sstages[slot][pl.ds(k * 16, 16)] = p & _IMASK
        pltpu.async_copy(g_hbm.at[sstages[slot]], rows[slot], gsems[slot])

    issue_gather(0, 0)
    issue_gather(1, 1)
    issue_gather(2, 2)

    def step(j, b):
        pltpu.make_async_copy(g_hbm.at[sstages[b]], rows[b],
                              gsems[b]).wait()
        for k in range(_CH // 16):
            p = ei_v[pl.ds(j * _CH + k * 16, 16)]
            dstage[pl.ds(k * 16, 16)] = lax.shift_right_logical(p, 14)
        pltpu.sync_copy(rows[b], acc_sh.at[dstage], add=True)

        @pl.when(j + 3 < kc)
        def _():
            issue_gather(j + 3, (b + 3) % 4)

    def quad(r, carry):
        j = 4 * r
        for b in range(4):
            step(j + b, b)
        return carry

    lax.fori_loop(0, kc // 4, quad, 0)

    plsc.subcore_barrier()
    base = s * 624
    pltpu.sync_copy(acc_sh.at[pl.ds(base, 624)],
                    out_hbm.at[c, pl.ds(base, 624)])

    @pl.when(s == _NS - 1)
    def _():
        pltpu.sync_copy(acc_sh.at[pl.ds(_NS * 624, _N - _NS * 624)],
                        out_hbm.at[c, pl.ds(_NS * 624, _N - _NS * 624)])


def _edge_pass(g, ep):
    kf = pl.kernel(
        _edge_body,
        out_type=jax.ShapeDtypeStruct((_NC, _N, _D), jnp.float32),
        mesh=plsc.VectorSubcoreMesh(core_axis_name="c", subcore_axis_name="s"),
        scratch_types=(
            [pltpu.VMEM((_EPWF,), jnp.int32)]
            + [pltpu.VMEM((_CH,), jnp.int32) for _ in range(5)]
            + [pltpu.VMEM((_CH, _D), jnp.float32) for _ in range(4)]
            + [pltpu.VMEM_SHARED((_NACC, _D), jnp.float32)]
            + [pltpu.SemaphoreType.DMA for _ in range(4)]
        ),
    )
    return kf(g, ep)


def _dinv_body(degp_ref, dinv_ref):
    deg = jnp.sum(degp_ref[...], axis=0) + 1.0
    dinv_ref[...] = lax.rsqrt(deg).reshape(_N, 1)


def _tc1_body(x_ref, w_ref, dinv_ref, g_ref):
    h = jnp.dot(x_ref[...], w_ref[...], preferred_element_type=jnp.float32)
    g_ref[...] = h * dinv_ref[...]


def _tc2_body(acc_ref, g1_ref, dinv_ref, b1_ref, w2_ref, g2_ref):
    dinv = dinv_ref[...]
    acc = acc_ref[0] + acc_ref[1]
    z = (acc + g1_ref[...]) * dinv + b1_ref[...]
    o = jnp.maximum(z, 0.0)
    h2 = jnp.dot(o, w2_ref[...], preferred_element_type=jnp.float32)
    g2_ref[...] = h2 * dinv


def _tc3_body(acc_ref, g2_ref, dinv_ref, b2_ref, wp_ref, bp_ref,
              h_ref, vals_ref, idx_ref):
    i = pl.program_id(0)
    dinv = dinv_ref[...]
    acc = acc_ref[0] + acc_ref[1]
    h = (acc + g2_ref[...]) * dinv + b2_ref[...]
    h_ref[...] = h
    pge = jnp.dot(h, wp_ref[...], preferred_element_type=jnp.float32) + bp_ref[...]
    m = jnp.max(pge, axis=0)[None, :]
    rows = lax.broadcasted_iota(jnp.int32, pge.shape, 0)
    am = jnp.min(jnp.where(pge == m, rows, _N), axis=0)[None, :] + i * _B

    @pl.when(i == 0)
    def _():
        vals_ref[...] = m
        idx_ref[...] = am

    @pl.when(i > 0)
    def _():
        cur = vals_ref[...]
        upd = m > cur
        vals_ref[...] = jnp.where(upd, m, cur)
        idx_ref[...] = jnp.where(upd, am, idx_ref[...])


def _row_spec():
    return pl.BlockSpec((_B, _D), lambda i: (i, 0))


def _full_spec(shape):
    nd = len(shape)
    return pl.BlockSpec(shape, lambda i: (0,) * nd)


def _dinv_spec():
    return pl.BlockSpec((_B, 1), lambda i: (i, 0))


def _acc_spec():
    return pl.BlockSpec((_NC, _B, _D), lambda i: (0, i, 0))


def kernel(x, edge_index, W1, b1, W2, b2, Wp, bp):
    src, dst = edge_index[0], edge_index[1]
    dst3 = dst.reshape(_NW, _DNCH, _DCH)
    pad = _EPAD - _E
    packed = jnp.bitwise_or(src, jnp.left_shift(dst, 14))
    ep = jnp.concatenate([packed, jnp.full((pad,), _N << 14, jnp.int32)])
    b1r = b1.reshape(1, _D)
    b2r = b2.reshape(1, _D)
    bpr = bp.reshape(1, _D)

    degp = _deg_counts(dst3).reshape(_NW, _N)

    dinv = pl.pallas_call(
        _dinv_body,
        grid=(1,),
        in_specs=[_full_spec((_NW, _N))],
        out_specs=_full_spec((_N, 1)),
        out_shape=jax.ShapeDtypeStruct((_N, 1), jnp.float32),
    )(degp)

    g1 = pl.pallas_call(
        _tc1_body,
        grid=(_G,),
        in_specs=[_row_spec(), _full_spec((_D, _D)), _dinv_spec()],
        out_specs=_row_spec(),
        out_shape=jax.ShapeDtypeStruct((_N, _D), jnp.float32),
    )(x, W1, dinv)

    acc1 = _edge_pass(g1, ep)

    g2 = pl.pallas_call(
        _tc2_body,
        grid=(_G,),
        in_specs=[_acc_spec(), _row_spec(), _dinv_spec(),
                  _full_spec((1, _D)), _full_spec((_D, _D))],
        out_specs=_row_spec(),
        out_shape=jax.ShapeDtypeStruct((_N, _D), jnp.float32),
    )(acc1, g1, dinv, b1r, W2)

    acc2 = _edge_pass(g2, ep)

    h, vals, idx = pl.pallas_call(
        _tc3_body,
        grid=(_G,),
        in_specs=[_acc_spec(), _row_spec(), _dinv_spec(),
                  _full_spec((1, _D)), _full_spec((_D, _D)),
                  _full_spec((1, _D))],
        out_specs=[_row_spec(), _full_spec((1, _D)), _full_spec((1, _D))],
        out_shape=[jax.ShapeDtypeStruct((_N, _D), jnp.float32),
                   jax.ShapeDtypeStruct((1, _D), jnp.float32),
                   jax.ShapeDtypeStruct((1, _D), jnp.int32)],
    )(acc2, g2, dinv, b2r, Wp, bpr)

    return h, vals.reshape(_D), idx.reshape(_D)
